# baseline (speedup 1.0000x reference)
import functools

import jax
import jax.numpy as jnp
from jax import lax
from jax.experimental import pallas as pl
from jax.experimental.pallas import tpu as pltpu
from jax.experimental.pallas import tpu_sc as plsc

N = 10000
E = 320000
DN = 128
DE = 16
DO = 128

NC = 2
NS = 16
L = 16
NW = NC * NS
EPW = E // NW
C = 80
NCHUNK = EPW // C
NPAD = 10240
RPT = NPAD // NS


def _when(cond):
    if isinstance(cond, bool):
        return (lambda f: f() if cond else None)
    return pl.when(cond)


def _h_body(x_ref, w_ref, b_ref, o_ref):
    o_ref[...] = lax.dot_general(
        x_ref[...], w_ref[...], (((1,), (1,)), ((), ())),
        preferred_element_type=jnp.float32) + b_ref[...]


def _project_h(x, w, b, block_rows):
    rows = x.shape[0]
    return pl.pallas_call(
        _h_body,
        grid=(rows // block_rows,),
        in_specs=[
            pl.BlockSpec((block_rows, x.shape[1]), lambda i: (i, 0)),
            pl.BlockSpec(w.shape, lambda i: (0, 0)),
            pl.BlockSpec(b.shape, lambda i: (0,)),
        ],
        out_specs=pl.BlockSpec((block_rows, w.shape[0]), lambda i: (i, 0)),
        out_shape=jax.ShapeDtypeStruct((rows, w.shape[0]), jnp.float32),
    )(x, w, b)


def _e_body(x_ref, w_ref, b_ref, o_ref):
    y = lax.dot_general(
        x_ref[...], w_ref[...], (((1,), (1,)), ((), ())),
        preferred_element_type=jnp.float32) + b_ref[...]
    o_ref[...] = y.reshape(o_ref.shape)


def _project_e(x, w, b, block_rows):
    rows = x.shape[0]
    bc = block_rows // C
    return pl.pallas_call(
        _e_body,
        grid=(rows // block_rows,),
        in_specs=[
            pl.BlockSpec((block_rows, x.shape[1]), lambda i: (i, 0)),
            pl.BlockSpec(w.shape, lambda i: (0, 0)),
            pl.BlockSpec(b.shape, lambda i: (0,)),
        ],
        out_specs=pl.BlockSpec((bc, C, DO), lambda i: (i, 0, 0)),
        out_shape=jax.ShapeDtypeStruct((rows // C, C, DO), jnp.float32),
    )(x, w, b)


def _sc_edge_kernel(h_hbm, e_hbm, src_hbm, dst_hbm, norm_hbm, zero_hbm,
                    part_hbm, src1, dst1, normb, hv0, hv1, ev0, ev1,
                    mv0, mv1, agg, gsem0, esem0, ssem0, sisem):
    c = lax.axis_index("c")
    s = lax.axis_index("s")
    wid = s * NC + c
    erow0 = wid * NCHUNK

    pltpu.sync_copy(zero_hbm.at[pl.ds(s * RPT, RPT)],
                    agg.at[pl.ds(s * RPT, RPT)])
    pltpu.sync_copy(norm_hbm.at[wid], normb)
    plsc.subcore_barrier()

    iz = wid * 0
    pltpu.sync_copy(src_hbm.at[wid].at[iz], src1)
    pltpu.async_copy(h_hbm.at[src1], hv0, gsem0)
    pltpu.async_copy(e_hbm.at[erow0 + iz], ev0, esem0)

    def chunk(i, carry):
        @_when(i >= 1)
        def _():
            pltpu.make_async_copy(mv0, agg.at[dst1], ssem0).wait()

        pltpu.sync_copy(dst_hbm.at[wid].at[i], dst1)
        pltpu.make_async_copy(h_hbm.at[src1], hv0, gsem0).wait()
        inx = jnp.minimum(i + 1, NCHUNK - 1)
        pltpu.async_copy(src_hbm.at[wid].at[inx], src1, sisem)
        pltpu.make_async_copy(e_hbm.at[erow0 + i], ev0, esem0).wait()

        def group(g, carry2):
            nv = normb[i, pl.ds(g * L, L)]
            for r16 in range(L):
                nrm = nv[r16]
                r = g * L + r16
                for j in range(DO // L):
                    x = hv0[r, pl.ds(j * L, L)] + ev0[r, pl.ds(j * L, L)]
                    mv0[r, pl.ds(j * L, L)] = jnp.maximum(x, 0.0) * nrm
            return carry2

        lax.fori_loop(0, C // L, group, 0)
        pltpu.async_copy(mv0, agg.at[dst1], ssem0, add=True)
        pltpu.make_async_copy(src_hbm.at[wid].at[i], src1, sisem).wait()
        pltpu.async_copy(h_hbm.at[src1], hv0, gsem0)
        pltpu.async_copy(e_hbm.at[erow0 + inx], ev0, esem0)
        return carry

    lax.fori_loop(0, NCHUNK, chunk, 0)
    pltpu.make_async_copy(mv0, agg.at[dst1], ssem0).wait()
    pltpu.make_async_copy(h_hbm.at[src1], hv0, gsem0).wait()
    pltpu.make_async_copy(e_hbm.at[erow0 + iz], ev0, esem0).wait()
    plsc.subcore_barrier()
    pltpu.sync_copy(agg.at[pl.ds(s * RPT, RPT)],
                    part_hbm.at[c, pl.ds(s * RPT, RPT)])


def _sc_messages(h, e3, src3, dst3, norm3, zero):
    mesh = plsc.VectorSubcoreMesh(core_axis_name="c", subcore_axis_name="s")
    kern = functools.partial(
        pl.kernel,
        mesh=mesh,
        out_type=jax.ShapeDtypeStruct((NC, NPAD, DO), jnp.float32),
        scratch_types=[
            pltpu.VMEM((C,), jnp.int32),
            pltpu.VMEM((C,), jnp.int32),
            pltpu.VMEM((NCHUNK, C), jnp.float32),
            pltpu.VMEM((C, DO), jnp.float32),
            pltpu.VMEM((C, DO), jnp.float32),
            pltpu.VMEM((C, DO), jnp.float32),
            pltpu.VMEM((C, DO), jnp.float32),
            pltpu.VMEM((C, DO), jnp.float32),
            pltpu.VMEM((C, DO), jnp.float32),
            pltpu.VMEM_SHARED((NPAD, DO), jnp.float32),
            pltpu.SemaphoreType.DMA,
            pltpu.SemaphoreType.DMA,
            pltpu.SemaphoreType.DMA,
            pltpu.SemaphoreType.DMA,
        ],
    )(_sc_edge_kernel)
    return kern(h, e3, src3, dst3, norm3, zero)


def _final_body(part_ref, h_ref, degs_ref, resw_ref, o_ref):
    res = jnp.maximum(h_ref[...] + resw_ref[...], 0.0) / degs_ref[...]
    o_ref[...] = part_ref[0] + part_ref[1] + res


def _finalize(part, h, degs, res_w, block_rows):
    return pl.pallas_call(
        _final_body,
        grid=(N // block_rows,),
        in_specs=[
            pl.BlockSpec((NC, block_rows, DO), lambda i: (0, i, 0)),
            pl.BlockSpec((block_rows, DO), lambda i: (i, 0)),
            pl.BlockSpec((block_rows, 1), lambda i: (i, 0)),
            pl.BlockSpec((1, DO), lambda i: (0, 0)),
        ],
        out_specs=pl.BlockSpec((block_rows, DO), lambda i: (i, 0)),
        out_shape=jax.ShapeDtypeStruct((N, DO), jnp.float32),
    )(part, h, degs, res_w)


def kernel(node_feats, edge_feats, degs, norm, edge_index, Wn, bn, We, be,
           res_w):
    h = _project_h(node_feats, Wn, bn, block_rows=2000)
    e3 = _project_e(edge_feats, We, be, block_rows=8000)
    src3 = edge_index[0].astype(jnp.int32).reshape(NW, NCHUNK, C)
    dst3 = edge_index[1].astype(jnp.int32).reshape(NW, NCHUNK, C)
    norm3 = norm.astype(jnp.float32).reshape(NW, NCHUNK, C)
    zero = jnp.zeros((NPAD, DO), jnp.float32)
    part = _sc_messages(h, e3, src3, dst3, norm3, zero)
    return _finalize(part, h, degs, res_w, block_rows=2000)


# speedup vs baseline: 1.0004x; 1.0004x over previous
import functools

import jax
import jax.numpy as jnp
from jax import lax
from jax.experimental import pallas as pl
from jax.experimental.pallas import tpu as pltpu
from jax.experimental.pallas import tpu_sc as plsc

N = 10000
E = 320000
DN = 128
DE = 16
DO = 128

NC = 2
NS = 16
L = 16
NW = NC * NS
EPW = E // NW
C = 80
NCHUNK = EPW // C
NPAD = 10240
RPT = NPAD // NS


def _when(cond):
    if isinstance(cond, bool):
        return (lambda f: f() if cond else None)
    return pl.when(cond)


def _h_body(x_ref, w_ref, b_ref, o_ref):
    o_ref[...] = lax.dot_general(
        x_ref[...], w_ref[...], (((1,), (1,)), ((), ())),
        preferred_element_type=jnp.float32) + b_ref[...]


def _project_h(x, w, b, block_rows):
    rows = x.shape[0]
    return pl.pallas_call(
        _h_body,
        grid=(rows // block_rows,),
        in_specs=[
            pl.BlockSpec((block_rows, x.shape[1]), lambda i: (i, 0)),
            pl.BlockSpec(w.shape, lambda i: (0, 0)),
            pl.BlockSpec(b.shape, lambda i: (0,)),
        ],
        out_specs=pl.BlockSpec((block_rows, w.shape[0]), lambda i: (i, 0)),
        out_shape=jax.ShapeDtypeStruct((rows, w.shape[0]), jnp.float32),
    )(x, w, b)


def _e_body(x_ref, w_ref, b_ref, o_ref):
    y = lax.dot_general(
        x_ref[...], w_ref[...], (((1,), (1,)), ((), ())),
        preferred_element_type=jnp.float32) + b_ref[...]
    o_ref[...] = y.reshape(o_ref.shape)


def _project_e(x, w, b, block_rows):
    rows = x.shape[0]
    bc = block_rows // C
    return pl.pallas_call(
        _e_body,
        grid=(rows // block_rows,),
        in_specs=[
            pl.BlockSpec((block_rows, x.shape[1]), lambda i: (i, 0)),
            pl.BlockSpec(w.shape, lambda i: (0, 0)),
            pl.BlockSpec(b.shape, lambda i: (0,)),
        ],
        out_specs=pl.BlockSpec((bc, C, DO), lambda i: (i, 0, 0)),
        out_shape=jax.ShapeDtypeStruct((rows // C, C, DO), jnp.float32),
    )(x, w, b)


def _sc_edge_kernel(h_hbm, e_hbm, src_hbm, dst_hbm, norm_hbm, zero_hbm,
                    part_hbm, src1, dst1, normb, hv0, hv1, ev0, ev1,
                    mv0, mv1, agg, gsem0, esem0, ssem0, sisem, dsem):
    c = lax.axis_index("c")
    s = lax.axis_index("s")
    wid = s * NC + c
    erow0 = wid * NCHUNK

    pltpu.sync_copy(zero_hbm.at[pl.ds(s * RPT, RPT)],
                    agg.at[pl.ds(s * RPT, RPT)])
    pltpu.sync_copy(norm_hbm.at[wid], normb)
    plsc.subcore_barrier()

    iz = wid * 0
    pltpu.sync_copy(src_hbm.at[wid].at[iz], src1)
    pltpu.async_copy(h_hbm.at[src1], hv0, gsem0)
    pltpu.async_copy(e_hbm.at[erow0 + iz], ev0, esem0)

    def chunk(i, carry):
        @_when(i >= 1)
        def _():
            pltpu.make_async_copy(mv0, agg.at[dst1], ssem0).wait()

        pltpu.async_copy(dst_hbm.at[wid].at[i], dst1, dsem)
        pltpu.make_async_copy(h_hbm.at[src1], hv0, gsem0).wait()
        inx = jnp.minimum(i + 1, NCHUNK - 1)
        pltpu.async_copy(src_hbm.at[wid].at[inx], src1, sisem)
        pltpu.make_async_copy(e_hbm.at[erow0 + i], ev0, esem0).wait()

        def group(g, carry2):
            nv = normb[i, pl.ds(g * L, L)]
            for r16 in range(L):
                nrm = nv[r16]
                r = g * L + r16
                for j in range(DO // L):
                    x = hv0[r, pl.ds(j * L, L)] + ev0[r, pl.ds(j * L, L)]
                    mv0[r, pl.ds(j * L, L)] = jnp.maximum(x, 0.0) * nrm
            return carry2

        lax.fori_loop(0, C // L, group, 0)
        pltpu.make_async_copy(dst_hbm.at[wid].at[i], dst1, dsem).wait()
        pltpu.async_copy(mv0, agg.at[dst1], ssem0, add=True)
        pltpu.make_async_copy(src_hbm.at[wid].at[i], src1, sisem).wait()
        pltpu.async_copy(h_hbm.at[src1], hv0, gsem0)
        pltpu.async_copy(e_hbm.at[erow0 + inx], ev0, esem0)
        return carry

    lax.fori_loop(0, NCHUNK, chunk, 0)
    pltpu.make_async_copy(mv0, agg.at[dst1], ssem0).wait()
    pltpu.make_async_copy(h_hbm.at[src1], hv0, gsem0).wait()
    pltpu.make_async_copy(e_hbm.at[erow0 + iz], ev0, esem0).wait()
    plsc.subcore_barrier()
    pltpu.sync_copy(agg.at[pl.ds(s * RPT, RPT)],
                    part_hbm.at[c, pl.ds(s * RPT, RPT)])


def _sc_messages(h, e3, src3, dst3, norm3, zero):
    mesh = plsc.VectorSubcoreMesh(core_axis_name="c", subcore_axis_name="s")
    kern = functools.partial(
        pl.kernel,
        mesh=mesh,
        out_type=jax.ShapeDtypeStruct((NC, NPAD, DO), jnp.float32),
        scratch_types=[
            pltpu.VMEM((C,), jnp.int32),
            pltpu.VMEM((C,), jnp.int32),
            pltpu.VMEM((NCHUNK, C), jnp.float32),
            pltpu.VMEM((C, DO), jnp.float32),
            pltpu.VMEM((C, DO), jnp.float32),
            pltpu.VMEM((C, DO), jnp.float32),
            pltpu.VMEM((C, DO), jnp.float32),
            pltpu.VMEM((C, DO), jnp.float32),
            pltpu.VMEM((C, DO), jnp.float32),
            pltpu.VMEM_SHARED((NPAD, DO), jnp.float32),
            pltpu.SemaphoreType.DMA,
            pltpu.SemaphoreType.DMA,
            pltpu.SemaphoreType.DMA,
            pltpu.SemaphoreType.DMA,
            pltpu.SemaphoreType.DMA,
        ],
    )(_sc_edge_kernel)
    return kern(h, e3, src3, dst3, norm3, zero)


def _final_body(part_ref, h_ref, degs_ref, resw_ref, o_ref):
    res = jnp.maximum(h_ref[...] + resw_ref[...], 0.0) / degs_ref[...]
    o_ref[...] = part_ref[0] + part_ref[1] + res


def _finalize(part, h, degs, res_w, block_rows):
    return pl.pallas_call(
        _final_body,
        grid=(N // block_rows,),
        in_specs=[
            pl.BlockSpec((NC, block_rows, DO), lambda i: (0, i, 0)),
            pl.BlockSpec((block_rows, DO), lambda i: (i, 0)),
            pl.BlockSpec((block_rows, 1), lambda i: (i, 0)),
            pl.BlockSpec((1, DO), lambda i: (0, 0)),
        ],
        out_specs=pl.BlockSpec((block_rows, DO), lambda i: (i, 0)),
        out_shape=jax.ShapeDtypeStruct((N, DO), jnp.float32),
    )(part, h, degs, res_w)


def kernel(node_feats, edge_feats, degs, norm, edge_index, Wn, bn, We, be,
           res_w):
    h = _project_h(node_feats, Wn, bn, block_rows=2000)
    e3 = _project_e(edge_feats, We, be, block_rows=8000)
    src3 = edge_index[0].astype(jnp.int32).reshape(NW, NCHUNK, C)
    dst3 = edge_index[1].astype(jnp.int32).reshape(NW, NCHUNK, C)
    norm3 = norm.astype(jnp.float32).reshape(NW, NCHUNK, C)
    zero = jnp.zeros((NPAD, DO), jnp.float32)
    part = _sc_messages(h, e3, src3, dst3, norm3, zero)
    return _finalize(part, h, degs, res_w, block_rows=2000)


# speedup vs baseline: 1.0075x; 1.0071x over previous
import functools

import jax
import jax.numpy as jnp
from jax import lax
from jax.experimental import pallas as pl
from jax.experimental.pallas import tpu as pltpu
from jax.experimental.pallas import tpu_sc as plsc

N = 10000
E = 320000
DN = 128
DE = 16
DO = 128

NC = 2
NS = 16
L = 16
NW = NC * NS
EPW = E // NW
C = 80
NCHUNK = EPW // C
NPAD = 10240
RPT = NPAD // NS


def _when(cond):
    if isinstance(cond, bool):
        return (lambda f: f() if cond else None)
    return pl.when(cond)


def _h_body(x_ref, w_ref, b_ref, o_ref):
    o_ref[...] = lax.dot_general(
        x_ref[...], w_ref[...], (((1,), (1,)), ((), ())),
        preferred_element_type=jnp.float32) + b_ref[...]


def _project_h(x, w, b, block_rows):
    rows = x.shape[0]
    return pl.pallas_call(
        _h_body,
        grid=(rows // block_rows,),
        in_specs=[
            pl.BlockSpec((block_rows, x.shape[1]), lambda i: (i, 0)),
            pl.BlockSpec(w.shape, lambda i: (0, 0)),
            pl.BlockSpec(b.shape, lambda i: (0,)),
        ],
        out_specs=pl.BlockSpec((block_rows, w.shape[0]), lambda i: (i, 0)),
        out_shape=jax.ShapeDtypeStruct((rows, w.shape[0]), jnp.float32),
    )(x, w, b)


def _e_body(x_ref, w_ref, b_ref, o_ref):
    y = lax.dot_general(
        x_ref[...], w_ref[...], (((1,), (1,)), ((), ())),
        preferred_element_type=jnp.float32) + b_ref[...]
    o_ref[...] = y.reshape(o_ref.shape)


def _project_e(x, w, b, block_rows):
    rows = x.shape[0]
    bc = block_rows // C
    return pl.pallas_call(
        _e_body,
        grid=(rows // block_rows,),
        in_specs=[
            pl.BlockSpec((block_rows, x.shape[1]), lambda i: (i, 0)),
            pl.BlockSpec(w.shape, lambda i: (0, 0)),
            pl.BlockSpec(b.shape, lambda i: (0,)),
        ],
        out_specs=pl.BlockSpec((bc, C, DO), lambda i: (i, 0, 0)),
        out_shape=jax.ShapeDtypeStruct((rows // C, C, DO), jnp.float32),
    )(x, w, b)


def _sc_edge_kernel(h_hbm, e_hbm, src_hbm, dst_hbm, norm_hbm, zero_hbm,
                    part_hbm, src1, dst1, normb, hv0, hv1, ev0, ev1,
                    mv0, mv1, agg, gsem0, esem0, ssem0, sisem, dsem):
    c = lax.axis_index("c")
    s = lax.axis_index("s")
    wid = s * NC + c
    erow0 = wid * NCHUNK

    pltpu.sync_copy(zero_hbm.at[pl.ds(s * RPT, RPT)],
                    agg.at[pl.ds(s * RPT, RPT)])
    pltpu.sync_copy(norm_hbm.at[wid], normb)
    plsc.subcore_barrier()

    iz = wid * 0
    pltpu.sync_copy(src_hbm.at[wid].at[iz], src1)
    pltpu.async_copy(h_hbm.at[src1], hv0, gsem0)
    pltpu.async_copy(e_hbm.at[erow0 + iz], ev0, esem0)

    def chunk(i, carry):
        @_when(i >= 1)
        def _():
            pltpu.make_async_copy(mv0, agg.at[dst1], ssem0).wait()

        pltpu.async_copy(dst_hbm.at[wid].at[i], dst1, dsem)
        pltpu.make_async_copy(h_hbm.at[src1], hv0, gsem0).wait()
        inx = jnp.minimum(i + 1, NCHUNK - 1)
        pltpu.async_copy(src_hbm.at[wid].at[inx], src1, sisem)
        pltpu.make_async_copy(e_hbm.at[erow0 + i], ev0, esem0).wait()

        def group(g, carry2):
            nv = normb[i, pl.ds(g * L, L)]
            for r16 in range(L):
                nrm = nv[r16]
                r = g * L + r16
                for j in range(DO // L):
                    x = hv0[r, pl.ds(j * L, L)] + ev0[r, pl.ds(j * L, L)]
                    mv0[r, pl.ds(j * L, L)] = jnp.maximum(x, 0.0) * nrm
            return carry2

        lax.fori_loop(0, C // L, group, 0)
        pltpu.make_async_copy(dst_hbm.at[wid].at[i], dst1, dsem).wait()
        pltpu.async_copy(mv0, agg.at[dst1], ssem0, add=True)
        pltpu.make_async_copy(src_hbm.at[wid].at[i], src1, sisem).wait()
        pltpu.async_copy(h_hbm.at[src1], hv0, gsem0)
        pltpu.async_copy(e_hbm.at[erow0 + inx], ev0, esem0)
        return carry

    lax.fori_loop(0, NCHUNK, chunk, 0)
    pltpu.make_async_copy(mv0, agg.at[dst1], ssem0).wait()
    pltpu.make_async_copy(h_hbm.at[src1], hv0, gsem0).wait()
    pltpu.make_async_copy(e_hbm.at[erow0 + iz], ev0, esem0).wait()
    plsc.subcore_barrier()
    pltpu.sync_copy(agg.at[pl.ds(s * RPT, RPT)],
                    part_hbm.at[c, pl.ds(s * RPT, RPT)])


def _sc_messages(h, e3, src3, dst3, norm3, zero):
    mesh = plsc.VectorSubcoreMesh(core_axis_name="c", subcore_axis_name="s")
    kern = functools.partial(
        pl.kernel,
        mesh=mesh,
        out_type=jax.ShapeDtypeStruct((NC, NPAD, DO), jnp.float32),
        scratch_types=[
            pltpu.VMEM((C,), jnp.int32),
            pltpu.VMEM((C,), jnp.int32),
            pltpu.VMEM((NCHUNK, C), jnp.float32),
            pltpu.VMEM((C, DO), jnp.float32),
            pltpu.VMEM((C, DO), jnp.float32),
            pltpu.VMEM((C, DO), jnp.float32),
            pltpu.VMEM((C, DO), jnp.float32),
            pltpu.VMEM((C, DO), jnp.float32),
            pltpu.VMEM((C, DO), jnp.float32),
            pltpu.VMEM_SHARED((NPAD, DO), jnp.float32),
            pltpu.SemaphoreType.DMA,
            pltpu.SemaphoreType.DMA,
            pltpu.SemaphoreType.DMA,
            pltpu.SemaphoreType.DMA,
            pltpu.SemaphoreType.DMA,
        ],
    )(_sc_edge_kernel)
    return kern(h, e3, src3, dst3, norm3, zero)


def _final_body(part_ref, h_ref, degs_ref, resw_ref, o_ref):
    res = jnp.maximum(h_ref[...] + resw_ref[...], 0.0) / degs_ref[...]
    o_ref[...] = part_ref[0] + part_ref[1] + res


def _finalize(part, h, degs, res_w, block_rows):
    return pl.pallas_call(
        _final_body,
        grid=(N // block_rows,),
        in_specs=[
            pl.BlockSpec((NC, block_rows, DO), lambda i: (0, i, 0)),
            pl.BlockSpec((block_rows, DO), lambda i: (i, 0)),
            pl.BlockSpec((block_rows, 1), lambda i: (i, 0)),
            pl.BlockSpec((1, DO), lambda i: (0, 0)),
        ],
        out_specs=pl.BlockSpec((block_rows, DO), lambda i: (i, 0)),
        out_shape=jax.ShapeDtypeStruct((N, DO), jnp.float32),
    )(part, h, degs, res_w)


def kernel(node_feats, edge_feats, degs, norm, edge_index, Wn, bn, We, be,
           res_w):
    h = _project_h(node_feats, Wn, bn, block_rows=10000)
    e3 = _project_e(edge_feats, We, be, block_rows=16000)
    src3 = edge_index[0].astype(jnp.int32).reshape(NW, NCHUNK, C)
    dst3 = edge_index[1].astype(jnp.int32).reshape(NW, NCHUNK, C)
    norm3 = norm.astype(jnp.float32).reshape(NW, NCHUNK, C)
    zero = jnp.zeros((NPAD, DO), jnp.float32)
    part = _sc_messages(h, e3, src3, dst3, norm3, zero)
    return _finalize(part, h, degs, res_w, block_rows=2000)


# speedup vs baseline: 1.0160x; 1.0085x over previous
import functools

import jax
import jax.numpy as jnp
from jax import lax
from jax.experimental import pallas as pl
from jax.experimental.pallas import tpu as pltpu
from jax.experimental.pallas import tpu_sc as plsc

N = 10000
E = 320000
DN = 128
DE = 16
DO = 128

NC = 2
NS = 16
L = 16
NW = NC * NS
EPW = E // NW
C = 80
NCHUNK = EPW // C
NPAD = 10240
RPT = NPAD // NS


def _when(cond):
    if isinstance(cond, bool):
        return (lambda f: f() if cond else None)
    return pl.when(cond)


def _h_body(x_ref, w_ref, b_ref, o_ref):
    o_ref[...] = lax.dot_general(
        x_ref[...], w_ref[...], (((1,), (1,)), ((), ())),
        preferred_element_type=jnp.float32) + b_ref[...]


def _project_h(x, w, b, block_rows):
    rows = x.shape[0]
    return pl.pallas_call(
        _h_body,
        grid=(rows // block_rows,),
        in_specs=[
            pl.BlockSpec((block_rows, x.shape[1]), lambda i: (i, 0)),
            pl.BlockSpec(w.shape, lambda i: (0, 0)),
            pl.BlockSpec(b.shape, lambda i: (0,)),
        ],
        out_specs=pl.BlockSpec((block_rows, w.shape[0]), lambda i: (i, 0)),
        out_shape=jax.ShapeDtypeStruct((rows, w.shape[0]), jnp.float32),
    )(x, w, b)


def _e_body(x_ref, w_ref, b_ref, o_ref):
    y = lax.dot_general(
        x_ref[...], w_ref[...], (((1,), (1,)), ((), ())),
        preferred_element_type=jnp.float32) + b_ref[...]
    o_ref[...] = y.reshape(o_ref.shape)


def _project_e(x, w, b, block_rows):
    rows = x.shape[0]
    bc = block_rows // C
    return pl.pallas_call(
        _e_body,
        grid=(rows // block_rows,),
        in_specs=[
            pl.BlockSpec((block_rows, x.shape[1]), lambda i: (i, 0)),
            pl.BlockSpec(w.shape, lambda i: (0, 0)),
            pl.BlockSpec(b.shape, lambda i: (0,)),
        ],
        out_specs=pl.BlockSpec((bc, C, DO), lambda i: (i, 0, 0)),
        out_shape=jax.ShapeDtypeStruct((rows // C, C, DO), jnp.float32),
    )(x, w, b)


def _sc_edge_kernel(h_hbm, e_hbm, src_hbm, dst_hbm, norm_hbm,
                    part_hbm, src1, dst1, normb, hv0, hv1, ev0, ev1,
                    mv0, mv1, agg, gsem0, esem0, ssem0, sisem, dsem):
    c = lax.axis_index("c")
    s = lax.axis_index("s")
    wid = s * NC + c
    erow0 = wid * NCHUNK

    def zrow(r, carry):
        for j in range(DO // L):
            mv0[r, pl.ds(j * L, L)] = jnp.zeros((L,), jnp.float32)
        return carry

    lax.fori_loop(0, C, zrow, 0)
    for t in range(RPT // C):
        pltpu.sync_copy(mv0, agg.at[pl.ds(s * RPT + t * C, C)])
    pltpu.sync_copy(norm_hbm.at[wid], normb)
    plsc.subcore_barrier()

    iz = wid * 0
    pltpu.sync_copy(src_hbm.at[wid].at[iz], src1)
    pltpu.async_copy(h_hbm.at[src1], hv0, gsem0)
    pltpu.async_copy(e_hbm.at[erow0 + iz], ev0, esem0)

    def chunk(i, carry):
        @_when(i >= 1)
        def _():
            pltpu.make_async_copy(mv0, agg.at[dst1], ssem0).wait()

        pltpu.async_copy(dst_hbm.at[wid].at[i], dst1, dsem)
        pltpu.make_async_copy(h_hbm.at[src1], hv0, gsem0).wait()
        inx = jnp.minimum(i + 1, NCHUNK - 1)
        pltpu.async_copy(src_hbm.at[wid].at[inx], src1, sisem)
        pltpu.make_async_copy(e_hbm.at[erow0 + i], ev0, esem0).wait()

        def group(g, carry2):
            nv = normb[i, pl.ds(g * L, L)]
            for r16 in range(L):
                nrm = nv[r16]
                r = g * L + r16
                for j in range(DO // L):
                    x = hv0[r, pl.ds(j * L, L)] + ev0[r, pl.ds(j * L, L)]
                    mv0[r, pl.ds(j * L, L)] = jnp.maximum(x, 0.0) * nrm
            return carry2

        lax.fori_loop(0, C // L, group, 0)
        pltpu.make_async_copy(dst_hbm.at[wid].at[i], dst1, dsem).wait()
        pltpu.async_copy(mv0, agg.at[dst1], ssem0, add=True)
        pltpu.make_async_copy(src_hbm.at[wid].at[i], src1, sisem).wait()
        pltpu.async_copy(h_hbm.at[src1], hv0, gsem0)
        pltpu.async_copy(e_hbm.at[erow0 + inx], ev0, esem0)
        return carry

    lax.fori_loop(0, NCHUNK, chunk, 0)
    pltpu.make_async_copy(mv0, agg.at[dst1], ssem0).wait()
    pltpu.make_async_copy(h_hbm.at[src1], hv0, gsem0).wait()
    pltpu.make_async_copy(e_hbm.at[erow0 + iz], ev0, esem0).wait()
    plsc.subcore_barrier()
    pltpu.sync_copy(agg.at[pl.ds(s * RPT, RPT)],
                    part_hbm.at[c, pl.ds(s * RPT, RPT)])


def _sc_messages(h, e3, src3, dst3, norm3):
    mesh = plsc.VectorSubcoreMesh(core_axis_name="c", subcore_axis_name="s")
    kern = functools.partial(
        pl.kernel,
        mesh=mesh,
        out_type=jax.ShapeDtypeStruct((NC, NPAD, DO), jnp.float32),
        scratch_types=[
            pltpu.VMEM((C,), jnp.int32),
            pltpu.VMEM((C,), jnp.int32),
            pltpu.VMEM((NCHUNK, C), jnp.float32),
            pltpu.VMEM((C, DO), jnp.float32),
            pltpu.VMEM((C, DO), jnp.float32),
            pltpu.VMEM((C, DO), jnp.float32),
            pltpu.VMEM((C, DO), jnp.float32),
            pltpu.VMEM((C, DO), jnp.float32),
            pltpu.VMEM((C, DO), jnp.float32),
            pltpu.VMEM_SHARED((NPAD, DO), jnp.float32),
            pltpu.SemaphoreType.DMA,
            pltpu.SemaphoreType.DMA,
            pltpu.SemaphoreType.DMA,
            pltpu.SemaphoreType.DMA,
            pltpu.SemaphoreType.DMA,
        ],
    )(_sc_edge_kernel)
    return kern(h, e3, src3, dst3, norm3)


def _final_body(part_ref, h_ref, degs_ref, resw_ref, o_ref):
    res = jnp.maximum(h_ref[...] + resw_ref[...], 0.0) / degs_ref[...]
    o_ref[...] = part_ref[0] + part_ref[1] + res


def _finalize(part, h, degs, res_w, block_rows):
    return pl.pallas_call(
        _final_body,
        grid=(N // block_rows,),
        in_specs=[
            pl.BlockSpec((NC, block_rows, DO), lambda i: (0, i, 0)),
            pl.BlockSpec((block_rows, DO), lambda i: (i, 0)),
            pl.BlockSpec((block_rows, 1), lambda i: (i, 0)),
            pl.BlockSpec((1, DO), lambda i: (0, 0)),
        ],
        out_specs=pl.BlockSpec((block_rows, DO), lambda i: (i, 0)),
        out_shape=jax.ShapeDtypeStruct((N, DO), jnp.float32),
    )(part, h, degs, res_w)


def kernel(node_feats, edge_feats, degs, norm, edge_index, Wn, bn, We, be,
           res_w):
    h = _project_h(node_feats, Wn, bn, block_rows=10000)
    e3 = _project_e(edge_feats, We, be, block_rows=16000)
    src3 = edge_index[0].astype(jnp.int32).reshape(NW, NCHUNK, C)
    dst3 = edge_index[1].astype(jnp.int32).reshape(NW, NCHUNK, C)
    norm3 = norm.astype(jnp.float32).reshape(NW, NCHUNK, C)
    part = _sc_messages(h, e3, src3, dst3, norm3)
    return _finalize(part, h, degs, res_w, block_rows=2000)
